# baseline (speedup 1.0000x reference)
import functools

import jax
import jax.numpy as jnp
from jax import lax
from jax.experimental import pallas as pl
from jax.experimental.pallas import tpu as pltpu
from jax.experimental.pallas import tpu_sc as plsc

NC, NS, L = 2, 16, 16
NW = NC * NS


def kernel(x, pe_weight):
    B, T, D = x.shape
    TPW = T // NW
    CT = 4
    NCH = TPW // CT
    VECS = D // L

    @functools.partial(
        pl.kernel,
        out_type=jax.ShapeDtypeStruct((B, T, D), jnp.float32),
        mesh=plsc.VectorSubcoreMesh(core_axis_name="c", subcore_axis_name="s"),
        scratch_types=[
            [[pltpu.VMEM((CT, D), jnp.float32) for _ in range(2)] for _ in range(B)],
            [pltpu.VMEM((CT, D), jnp.float32) for _ in range(2)],
            [[pltpu.SemaphoreType.DMA for _ in range(2)] for _ in range(B)],
            [[pltpu.SemaphoreType.DMA for _ in range(2)] for _ in range(B)],
            [pltpu.SemaphoreType.DMA for _ in range(2)],
        ],
    )
    def sc_add_pe(x_hbm, pe_hbm, out_hbm, xbufs, pebufs, sin, sout, spe):
        wid = lax.axis_index("s") * NC + lax.axis_index("c")
        base = wid * TPW

        def x_src(c, b):
            return x_hbm.at[b, pl.ds(base + c * CT, CT)]

        def o_dst(c, b):
            return out_hbm.at[b, pl.ds(base + c * CT, CT)]

        def pe_src(c):
            return pe_hbm.at[pl.ds(base + c * CT, CT)]

        for P in range(2):
            pltpu.async_copy(pe_src(P), pebufs[P], spe[P])
            for b in range(B):
                pltpu.async_copy(x_src(P, b), xbufs[b][P], sin[b][P])

        def add_chunk(xb, pb):
            @plsc.parallel_loop(0, CT)
            def _rows(r):
                @plsc.parallel_loop(0, VECS, unroll=32)
                def _vecs(i):
                    sl = pl.ds(i * L, L)
                    plsc.addupdate(xb.at[r, sl], pb[r, sl])

        def compute_chunk(c, P):
            pltpu.make_async_copy(pe_src(c), pebufs[P], spe[P]).wait()
            for b in range(B):
                pltpu.make_async_copy(x_src(c, b), xbufs[b][P], sin[b][P]).wait()
                add_chunk(xbufs[b][P], pebufs[P])
                pltpu.async_copy(xbufs[b][P], o_dst(c, b), sout[b][P])

            @pl.when(c + 2 < NCH)
            def _():
                pltpu.async_copy(pe_src(c + 2), pebufs[P], spe[P])

        def recycle_chunk(c, P):
            for b in range(B):
                pltpu.make_async_copy(xbufs[b][P], o_dst(c, b), sout[b][P]).wait()

                @pl.when(c + 2 < NCH)
                def _():
                    pltpu.async_copy(x_src(c + 2, b), xbufs[b][P], sin[b][P])

        def g_body(g, carry):
            c0 = 2 * g
            compute_chunk(c0, 0)
            compute_chunk(c0 + 1, 1)
            recycle_chunk(c0, 0)
            recycle_chunk(c0 + 1, 1)
            return carry

        lax.fori_loop(0, NCH // 2, g_body, 0)

    return sc_add_pe(x, pe_weight)


# speedup vs baseline: 1.0153x; 1.0153x over previous
import functools

import jax
import jax.numpy as jnp
from jax import lax
from jax.experimental import pallas as pl
from jax.experimental.pallas import tpu as pltpu
from jax.experimental.pallas import tpu_sc as plsc

NC, NS, L = 2, 16, 16
NW = NC * NS


def kernel(x, pe_weight):
    B, T, D = x.shape
    TPW = T // NW
    CT = 4
    NCH = TPW // CT
    VECS = D // L

    @functools.partial(
        pl.kernel,
        out_type=jax.ShapeDtypeStruct((B, T, D), jnp.float32),
        mesh=plsc.VectorSubcoreMesh(core_axis_name="c", subcore_axis_name="s"),
        scratch_types=[
            [[pltpu.VMEM((CT, D), jnp.float32) for _ in range(2)] for _ in range(B)],
            [pltpu.VMEM((CT, D), jnp.float32) for _ in range(2)],
            [[pltpu.SemaphoreType.DMA for _ in range(2)] for _ in range(B)],
            [[pltpu.SemaphoreType.DMA for _ in range(2)] for _ in range(B)],
            [pltpu.SemaphoreType.DMA for _ in range(2)],
        ],
    )
    def sc_add_pe(x_hbm, pe_hbm, out_hbm, xbufs, pebufs, sin, sout, spe):
        wid = lax.axis_index("s") * NC + lax.axis_index("c")
        base = wid * TPW

        def x_src(c, b):
            return x_hbm.at[b, pl.ds(base + c * CT, CT)]

        def o_dst(c, b):
            return out_hbm.at[b, pl.ds(base + c * CT, CT)]

        def pe_src(c):
            return pe_hbm.at[pl.ds(base + c * CT, CT)]

        for P in range(2):
            pltpu.async_copy(pe_src(P), pebufs[P], spe[P])
            for b in range(B):
                pltpu.async_copy(x_src(P, b), xbufs[b][P], sin[b][P])

        def add_chunk(xb, pb):
            @plsc.parallel_loop(0, CT)
            def _rows(r):
                @plsc.parallel_loop(0, VECS, unroll=8)
                def _vecs(i):
                    sl = pl.ds(i * L, L)
                    xb[r, sl] = xb[r, sl] + pb[r, sl]

        def compute_chunk(c, P):
            pltpu.make_async_copy(pe_src(c), pebufs[P], spe[P]).wait()
            for b in range(B):
                pltpu.make_async_copy(x_src(c, b), xbufs[b][P], sin[b][P]).wait()
                add_chunk(xbufs[b][P], pebufs[P])
                pltpu.async_copy(xbufs[b][P], o_dst(c, b), sout[b][P])

            @pl.when(c + 2 < NCH)
            def _():
                pltpu.async_copy(pe_src(c + 2), pebufs[P], spe[P])

        def recycle_chunk(c, P):
            for b in range(B):
                pltpu.make_async_copy(xbufs[b][P], o_dst(c, b), sout[b][P]).wait()

                @pl.when(c + 2 < NCH)
                def _():
                    pltpu.async_copy(x_src(c + 2, b), xbufs[b][P], sin[b][P])

        def g_body(g, carry):
            c0 = 2 * g
            compute_chunk(c0, 0)
            compute_chunk(c0 + 1, 1)
            recycle_chunk(c0, 0)
            recycle_chunk(c0 + 1, 1)
            return carry

        lax.fori_loop(0, NCH // 2, g_body, 0)

    return sc_add_pe(x, pe_weight)
